# baseline (speedup 1.0000x reference)
import jax
import jax.numpy as jnp
from jax.experimental import pallas as pl
from jax.experimental.pallas import tpu as pltpu

_BLK = 2048


def _weight_block(w2_ref, out_ref):
    w2s = w2_ref[0, 0]
    w1 = jnp.zeros((_BLK, 1), jnp.float32)
    w3 = w1 * w2s + w1
    sq = w3 * w3
    w = w3 * jax.lax.rsqrt(jnp.maximum(sq, jnp.float32(1e-12)))
    out_ref[...] = jnp.broadcast_to(w, out_ref.shape)


def kernel(x, w2):
    b, length = x.shape
    rows = b * length // 128
    out = pl.pallas_call(
        _weight_block,
        grid=(rows // _BLK,),
        in_specs=[pl.BlockSpec((1, 1), lambda i: (0, 0))],
        out_specs=pl.BlockSpec((_BLK, 128), lambda i: (i, 0)),
        out_shape=jax.ShapeDtypeStruct((rows, 128), jnp.float32),
        compiler_params=pltpu.CompilerParams(
            dimension_semantics=("parallel",)),
    )(w2.reshape(1, 1))
    return out.reshape(b, length, 1)


# speedup vs baseline: 1.4772x; 1.4772x over previous
import jax
import jax.numpy as jnp
from jax.experimental import pallas as pl
from jax.experimental.pallas import tpu as pltpu

_BLK = 8192


def _weight_block(w2_ref, out_ref):
    w2s = w2_ref[0, 0]
    w1 = jnp.zeros((_BLK, 1), jnp.float32)
    w3 = w1 * w2s + w1
    sq = w3 * w3
    w = w3 * jax.lax.rsqrt(jnp.maximum(sq, jnp.float32(1e-12)))
    out_ref[...] = jnp.broadcast_to(w, out_ref.shape)


def kernel(x, w2):
    b, length = x.shape
    rows = b * length // 128
    out = pl.pallas_call(
        _weight_block,
        grid=(rows // _BLK,),
        in_specs=[pl.BlockSpec((1, 1), lambda i: (0, 0))],
        out_specs=pl.BlockSpec((_BLK, 128), lambda i: (i, 0)),
        out_shape=jax.ShapeDtypeStruct((rows, 128), jnp.float32),
        compiler_params=pltpu.CompilerParams(
            dimension_semantics=("parallel",)),
    )(w2.reshape(1, 1))
    return out.reshape(b, length, 1)


# speedup vs baseline: 1.4823x; 1.0035x over previous
import jax
import jax.numpy as jnp
from jax.experimental import pallas as pl
from jax.experimental.pallas import tpu as pltpu

_BLK = 8192


def _weight_block(w2_ref, out_ref):
    w2s = w2_ref[0]
    w1 = jnp.zeros((_BLK, 1), jnp.float32)
    w3 = w1 * w2s + w1
    sq = w3 * w3
    w = w3 * jax.lax.rsqrt(jnp.maximum(sq, jnp.float32(1e-12)))
    out_ref[...] = jnp.broadcast_to(w, out_ref.shape)


def kernel(x, w2):
    b, length = x.shape
    rows = b * length // 128
    out = pl.pallas_call(
        _weight_block,
        grid=(rows // _BLK,),
        in_specs=[pl.BlockSpec(memory_space=pltpu.MemorySpace.SMEM)],
        out_specs=pl.BlockSpec((_BLK, 128), lambda i: (i, 0)),
        out_shape=jax.ShapeDtypeStruct((rows, 128), jnp.float32),
        compiler_params=pltpu.CompilerParams(
            dimension_semantics=("parallel",)),
    )(w2.reshape(1))
    return out.reshape(b, length, 1)
